# baseline (device time: 200559 ns/iter reference)
import jax
import jax.numpy as jnp
from jax import lax
from jax.experimental import pallas as pl
from jax.experimental.pallas import tpu as pltpu

N_DEV = 8
H_PER = 8
DH = 128
SCALE = 0.08838834764831843


def kernel(x, Wq, Wo, K_ext, V_ext):
    _, sq, dm = x.shape
    skv = K_ext.shape[1]

    i = lax.axis_index("i")
    x2 = x[0].astype(jnp.bfloat16)
    wq = Wq.astype(jnp.bfloat16)
    wo = Wo.astype(jnp.bfloat16)
    k = lax.dynamic_slice_in_dim(K_ext[0], i * H_PER, H_PER, axis=1)
    v = lax.dynamic_slice_in_dim(V_ext[0], i * H_PER, H_PER, axis=1)
    k = k.transpose(1, 0, 2).astype(jnp.bfloat16)
    v = v.transpose(1, 0, 2).astype(jnp.bfloat16)
    v = jnp.concatenate(
        [v,
         jnp.ones((H_PER, skv, 1), jnp.bfloat16),
         jnp.zeros((H_PER, skv, DH - 1), jnp.bfloat16)], axis=2)

    def body(x_ref, wq_ref, wo_ref, k_ref, v_ref, out_ref,
             x_buf, acc_buf, q_scr, o_scr, own_scr,
             x_ssem, x_rsem, a_ssem, a_rsem):
        my = lax.axis_index("i")
        left = lax.rem(my + N_DEV - 1, N_DEV)
        right = lax.rem(my + 1, N_DEV)

        barrier = pltpu.get_barrier_semaphore()
        for nbr in (left, right):
            pl.semaphore_signal(barrier, inc=1, device_id=(nbr,),
                                device_id_type=pl.DeviceIdType.MESH)
        pl.semaphore_wait(barrier, 2)

        def compute_contrib(x_src):
            q = lax.dot_general(x_src, wq_ref[...], (((1,), (0,)), ((), ())),
                                preferred_element_type=jnp.float32) * SCALE
            q_scr[...] = q.astype(jnp.bfloat16)

            def hbody(h, carry):
                qh = q_scr[:, pl.ds(h * DH, DH)]
                kh = k_ref[h]
                s = lax.dot_general(qh, kh, (((1,), (1,)), ((), ())),
                                    preferred_element_type=jnp.float32)
                p = jnp.exp(s).astype(jnp.bfloat16)
                pv = lax.dot_general(p, v_ref[h], (((1,), (0,)), ((), ())),
                                     preferred_element_type=jnp.float32)
                oh = pv[:, :DH]
                l = pv[:, DH:DH + 1]
                o_scr[:, pl.ds(h * DH, DH)] = (oh / l).astype(jnp.bfloat16)
                return carry

            lax.fori_loop(0, H_PER, hbody, 0)
            return lax.dot_general(o_scr[...], wo_ref[...],
                                   (((1,), (0,)), ((), ())),
                                   preferred_element_type=jnp.float32)

        def x_send(t):
            return pltpu.make_async_remote_copy(
                src_ref=x_ref if t == 0 else x_buf.at[t - 1],
                dst_ref=x_buf.at[t],
                send_sem=x_ssem.at[t], recv_sem=x_rsem.at[t],
                device_id=(right,), device_id_type=pl.DeviceIdType.MESH)

        def a_send(t):
            return pltpu.make_async_remote_copy(
                src_ref=acc_buf.at[t - 1],
                dst_ref=acc_buf.at[t],
                send_sem=a_ssem.at[t - 1], recv_sem=a_rsem.at[t - 1],
                device_id=(right,), device_id_type=pl.DeviceIdType.MESH)

        pending = []
        x_rdmas = [x_send(t) for t in range(N_DEV - 1)]
        a_rdmas = [None] + [a_send(t) for t in range(1, N_DEV)]

        x_rdmas[0].start()
        pending.append(x_rdmas[0])
        own_scr[...] = compute_contrib(x_ref[...])

        for t in range(N_DEV - 1):
            x_rdmas[t].wait_recv()
            if t + 1 < N_DEV - 1:
                x_rdmas[t + 1].start()
                pending.append(x_rdmas[t + 1])

            c = compute_contrib(x_buf[t])
            if t == 0:
                acc_buf[0] = c
            else:
                a_rdmas[t].wait_recv()
                acc_buf[t] += c
            a_rdmas[t + 1].start()
            pending.append(a_rdmas[t + 1])

        a_rdmas[N_DEV - 1].wait_recv()
        out_ref[...] = acc_buf[N_DEV - 1] + own_scr[...]

        for r in pending:
            r.wait_send()

    out = pl.pallas_call(
        body,
        out_shape=jax.ShapeDtypeStruct((sq, dm), jnp.float32),
        in_specs=[pl.BlockSpec(memory_space=pltpu.VMEM)] * 5,
        out_specs=pl.BlockSpec(memory_space=pltpu.VMEM),
        scratch_shapes=[
            pltpu.VMEM((N_DEV - 1, sq, dm), jnp.bfloat16),
            pltpu.VMEM((N_DEV, sq, dm), jnp.float32),
            pltpu.VMEM((sq, dm), jnp.bfloat16),
            pltpu.VMEM((sq, dm), jnp.bfloat16),
            pltpu.VMEM((sq, dm), jnp.float32),
            pltpu.SemaphoreType.DMA((N_DEV - 1,)),
            pltpu.SemaphoreType.DMA((N_DEV - 1,)),
            pltpu.SemaphoreType.DMA((N_DEV - 1,)),
            pltpu.SemaphoreType.DMA((N_DEV - 1,)),
        ],
        compiler_params=pltpu.CompilerParams(collective_id=0),
    )(x2, wq, wo, k, v)

    return out.reshape(1, sq, dm)


# device time: 180120 ns/iter; 1.1135x vs baseline; 1.1135x over previous
import jax
import jax.numpy as jnp
from jax import lax
from jax.experimental import pallas as pl
from jax.experimental.pallas import tpu as pltpu

N_DEV = 8
H_PER = 8
DH = 128
SCALE = 0.08838834764831843


def kernel(x, Wq, Wo, K_ext, V_ext):
    _, sq, dm = x.shape
    skv = K_ext.shape[1]

    i = lax.axis_index("i")
    x2 = x[0].astype(jnp.bfloat16)
    wq = Wq.astype(jnp.bfloat16)
    wo = Wo.astype(jnp.bfloat16)
    k = lax.dynamic_slice_in_dim(K_ext[0], i * H_PER, H_PER, axis=1)
    v = lax.dynamic_slice_in_dim(V_ext[0], i * H_PER, H_PER, axis=1)
    k = k.astype(jnp.bfloat16).transpose(1, 0, 2)
    v = v.astype(jnp.bfloat16).transpose(1, 0, 2)

    def body(x_ref, wq_ref, wo_ref, k_ref, v_ref, out_ref,
             x_buf, acc_buf, q_scr, o_scr, own_scr,
             x_ssem, x_rsem, a_ssem, a_rsem):
        my = lax.axis_index("i")
        left = lax.rem(my + N_DEV - 1, N_DEV)
        right = lax.rem(my + 1, N_DEV)

        barrier = pltpu.get_barrier_semaphore()
        for nbr in (left, right):
            pl.semaphore_signal(barrier, inc=1, device_id=(nbr,),
                                device_id_type=pl.DeviceIdType.MESH)
        pl.semaphore_wait(barrier, 2)

        def compute_contrib(x_src):
            q = lax.dot_general(x_src, wq_ref[...], (((1,), (0,)), ((), ())),
                                preferred_element_type=jnp.float32) * SCALE
            q_scr[...] = q.astype(jnp.bfloat16)

            for h in range(H_PER):
                qh = q_scr[:, h * DH:(h + 1) * DH]
                kh = k_ref[h]
                s = lax.dot_general(qh, kh, (((1,), (1,)), ((), ())),
                                    preferred_element_type=jnp.float32)
                p = jnp.exp(s)
                l = jnp.sum(p, axis=1, keepdims=True)
                oh = lax.dot_general(p.astype(jnp.bfloat16), v_ref[h],
                                     (((1,), (0,)), ((), ())),
                                     preferred_element_type=jnp.float32)
                o_scr[:, h * DH:(h + 1) * DH] = (oh / l).astype(jnp.bfloat16)
            return lax.dot_general(o_scr[...], wo_ref[...],
                                   (((1,), (0,)), ((), ())),
                                   preferred_element_type=jnp.float32)

        def x_send(t):
            return pltpu.make_async_remote_copy(
                src_ref=x_ref if t == 0 else x_buf.at[t - 1],
                dst_ref=x_buf.at[t],
                send_sem=x_ssem.at[t], recv_sem=x_rsem.at[t],
                device_id=(right,), device_id_type=pl.DeviceIdType.MESH)

        def a_send(t):
            return pltpu.make_async_remote_copy(
                src_ref=acc_buf.at[t - 1],
                dst_ref=acc_buf.at[t],
                send_sem=a_ssem.at[t - 1], recv_sem=a_rsem.at[t - 1],
                device_id=(right,), device_id_type=pl.DeviceIdType.MESH)

        pending = []
        x_rdmas = [x_send(t) for t in range(N_DEV - 1)]
        a_rdmas = [None] + [a_send(t) for t in range(1, N_DEV)]

        x_rdmas[0].start()
        pending.append(x_rdmas[0])
        own_scr[...] = compute_contrib(x_ref[...])

        for t in range(N_DEV - 1):
            x_rdmas[t].wait_recv()
            if t + 1 < N_DEV - 1:
                x_rdmas[t + 1].start()
                pending.append(x_rdmas[t + 1])

            c = compute_contrib(x_buf[t])
            if t == 0:
                acc_buf[0] = c
            else:
                a_rdmas[t].wait_recv()
                acc_buf[t] += c
            a_rdmas[t + 1].start()
            pending.append(a_rdmas[t + 1])

        a_rdmas[N_DEV - 1].wait_recv()
        out_ref[...] = acc_buf[N_DEV - 1] + own_scr[...]

        for r in pending:
            r.wait_send()

    out = pl.pallas_call(
        body,
        out_shape=jax.ShapeDtypeStruct((sq, dm), jnp.float32),
        in_specs=[pl.BlockSpec(memory_space=pltpu.VMEM)] * 5,
        out_specs=pl.BlockSpec(memory_space=pltpu.VMEM),
        scratch_shapes=[
            pltpu.VMEM((N_DEV - 1, sq, dm), jnp.bfloat16),
            pltpu.VMEM((N_DEV, sq, dm), jnp.float32),
            pltpu.VMEM((sq, dm), jnp.bfloat16),
            pltpu.VMEM((sq, dm), jnp.bfloat16),
            pltpu.VMEM((sq, dm), jnp.float32),
            pltpu.SemaphoreType.DMA((N_DEV - 1,)),
            pltpu.SemaphoreType.DMA((N_DEV - 1,)),
            pltpu.SemaphoreType.DMA((N_DEV - 1,)),
            pltpu.SemaphoreType.DMA((N_DEV - 1,)),
        ],
        compiler_params=pltpu.CompilerParams(collective_id=0),
    )(x2, wq, wo, k, v)

    return out.reshape(1, sq, dm)


# device time: 147555 ns/iter; 1.3592x vs baseline; 1.2207x over previous
import jax
import jax.numpy as jnp
from jax import lax
from jax.experimental import pallas as pl
from jax.experimental.pallas import tpu as pltpu

N_DEV = 8
H_PER = 8
DH = 128
SCALE = 0.08838834764831843


def kernel(x, Wq, Wo, K_ext, V_ext):
    _, sq, dm = x.shape
    skv = K_ext.shape[1]

    i = lax.axis_index("i")
    x2 = x[0].astype(jnp.bfloat16)
    wq = Wq.astype(jnp.bfloat16)
    wo = Wo.astype(jnp.bfloat16)
    k = lax.dynamic_slice_in_dim(K_ext[0], i * H_PER, H_PER, axis=1)
    v = lax.dynamic_slice_in_dim(V_ext[0], i * H_PER, H_PER, axis=1)
    k = k.astype(jnp.bfloat16).transpose(1, 0, 2)
    v = v.astype(jnp.bfloat16).transpose(1, 0, 2)

    def body(x_ref, wq_ref, wo_ref, k_ref, v_ref, out_ref,
             x_buf, acc_buf, q_scr, o_scr, own_scr,
             x_ssem, x_rsem, a_ssem, a_rsem):
        my = lax.axis_index("i")
        left = lax.rem(my + N_DEV - 1, N_DEV)
        right = lax.rem(my + 1, N_DEV)

        barrier = pltpu.get_barrier_semaphore()
        for nbr in (left, right):
            pl.semaphore_signal(barrier, inc=1, device_id=(nbr,),
                                device_id_type=pl.DeviceIdType.MESH)
        pl.semaphore_wait(barrier, 2)

        def compute_contrib(x_src):
            q = lax.dot_general(x_src, wq_ref[...], (((1,), (0,)), ((), ())),
                                preferred_element_type=jnp.float32) * SCALE
            q_scr[...] = q.astype(jnp.bfloat16)

            for h in range(H_PER):
                qh = q_scr[:, h * DH:(h + 1) * DH]
                kh = k_ref[h]
                s = lax.dot_general(qh, kh, (((1,), (1,)), ((), ())),
                                    preferred_element_type=jnp.float32)
                p = jnp.exp(s)
                l = jnp.sum(p, axis=1, keepdims=True)
                oh = lax.dot_general(p.astype(jnp.bfloat16), v_ref[h],
                                     (((1,), (0,)), ((), ())),
                                     preferred_element_type=jnp.float32)
                o_scr[:, h * DH:(h + 1) * DH] = (oh / l).astype(jnp.bfloat16)
            return lax.dot_general(o_scr[...], wo_ref[...],
                                   (((1,), (0,)), ((), ())),
                                   preferred_element_type=jnp.float32)

        def x_send(t):
            return pltpu.make_async_remote_copy(
                src_ref=x_ref if t == 0 else x_buf.at[t - 1],
                dst_ref=x_buf.at[t],
                send_sem=x_ssem.at[t], recv_sem=x_rsem.at[t],
                device_id=(right,), device_id_type=pl.DeviceIdType.MESH)

        def a_send(t):
            return pltpu.make_async_remote_copy(
                src_ref=acc_buf.at[t - 1],
                dst_ref=acc_buf.at[t],
                send_sem=a_ssem.at[t - 1], recv_sem=a_rsem.at[t - 1],
                device_id=(right,), device_id_type=pl.DeviceIdType.MESH)

        pending = []
        x_rdmas = [x_send(t) for t in range(N_DEV - 1)]
        a_rdmas = [None] + [a_send(t) for t in range(1, N_DEV)]

        x_rdmas[0].start()
        pending.append(x_rdmas[0])
        own_scr[...] = compute_contrib(x_ref[...])

        for t in range(N_DEV - 1):
            x_rdmas[t].wait_recv()
            if t + 1 < N_DEV - 1:
                x_rdmas[t + 1].start()
                pending.append(x_rdmas[t + 1])

            c = compute_contrib(x_buf[t])
            if t == 0:
                acc_buf[0] = c.astype(jnp.bfloat16)
            else:
                a_rdmas[t].wait_recv()
                acc_buf[t] = (acc_buf[t].astype(jnp.float32)
                              + c).astype(jnp.bfloat16)
            a_rdmas[t + 1].start()
            pending.append(a_rdmas[t + 1])

        a_rdmas[N_DEV - 1].wait_recv()
        out_ref[...] = acc_buf[N_DEV - 1].astype(jnp.float32) + own_scr[...]

        for r in pending:
            r.wait_send()

    out = pl.pallas_call(
        body,
        out_shape=jax.ShapeDtypeStruct((sq, dm), jnp.float32),
        in_specs=[pl.BlockSpec(memory_space=pltpu.VMEM)] * 5,
        out_specs=pl.BlockSpec(memory_space=pltpu.VMEM),
        scratch_shapes=[
            pltpu.VMEM((N_DEV - 1, sq, dm), jnp.bfloat16),
            pltpu.VMEM((N_DEV, sq, dm), jnp.bfloat16),
            pltpu.VMEM((sq, dm), jnp.bfloat16),
            pltpu.VMEM((sq, dm), jnp.bfloat16),
            pltpu.VMEM((sq, dm), jnp.float32),
            pltpu.SemaphoreType.DMA((N_DEV - 1,)),
            pltpu.SemaphoreType.DMA((N_DEV - 1,)),
            pltpu.SemaphoreType.DMA((N_DEV - 1,)),
            pltpu.SemaphoreType.DMA((N_DEV - 1,)),
        ],
        compiler_params=pltpu.CompilerParams(collective_id=0),
    )(x2, wq, wo, k, v)

    return out.reshape(1, sq, dm)


# device time: 128282 ns/iter; 1.5634x vs baseline; 1.1502x over previous
import jax
import jax.numpy as jnp
from jax import lax
from jax.experimental import pallas as pl
from jax.experimental.pallas import tpu as pltpu

N_DEV = 8
H_PER = 8
DH = 128
SCALE = 0.08838834764831843


def kernel(x, Wq, Wo, K_ext, V_ext):
    _, sq, dm = x.shape
    skv = K_ext.shape[1]

    x2 = x[0].astype(jnp.bfloat16)
    wq = Wq.astype(jnp.bfloat16)
    wo = Wo.astype(jnp.bfloat16)

    def body(x_ref, wq_ref, wo_ref, k_any, v_any, out_ref,
             k_ref, v_ref, stage, x_buf, acc_buf, q_scr, o_scr, own_scr,
             kv_sem, x_ssem, x_rsem, a_ssem, a_rsem):
        my = lax.axis_index("i")
        left = lax.rem(my + N_DEV - 1, N_DEV)
        right = lax.rem(my + 1, N_DEV)

        barrier = pltpu.get_barrier_semaphore()
        for nbr in (left, right):
            pl.semaphore_signal(barrier, inc=1, device_id=(nbr,),
                                device_id_type=pl.DeviceIdType.MESH)
        pl.semaphore_wait(barrier, 2)

        N_KV = 2 * H_PER

        def kv_dma(idx):
            src = k_any if idx < H_PER else v_any
            return pltpu.make_async_copy(
                src.at[0, :, my * H_PER + (idx % H_PER), :],
                stage.at[idx % 4],
                kv_sem.at[idx % 4])

        for idx in range(4):
            kv_dma(idx).start()
        for idx in range(N_KV):
            kv_dma(idx).wait()
            tgt = k_ref if idx < H_PER else v_ref
            tgt[idx % H_PER] = stage[idx % 4].astype(jnp.bfloat16)
            if idx + 4 < N_KV:
                kv_dma(idx + 4).start()

        def compute_contrib(x_src):
            q = lax.dot_general(x_src, wq_ref[...], (((1,), (0,)), ((), ())),
                                preferred_element_type=jnp.float32) * SCALE
            q_scr[...] = q.astype(jnp.bfloat16)

            for h in range(H_PER):
                qh = q_scr[:, h * DH:(h + 1) * DH]
                kh = k_ref[h]
                s = lax.dot_general(qh, kh, (((1,), (1,)), ((), ())),
                                    preferred_element_type=jnp.float32)
                p = jnp.exp(s)
                l = jnp.sum(p, axis=1, keepdims=True)
                oh = lax.dot_general(p.astype(jnp.bfloat16), v_ref[h],
                                     (((1,), (0,)), ((), ())),
                                     preferred_element_type=jnp.float32)
                o_scr[:, h * DH:(h + 1) * DH] = (oh / l).astype(jnp.bfloat16)
            return lax.dot_general(o_scr[...], wo_ref[...],
                                   (((1,), (0,)), ((), ())),
                                   preferred_element_type=jnp.float32)

        def x_send(t):
            return pltpu.make_async_remote_copy(
                src_ref=x_ref if t == 0 else x_buf.at[t - 1],
                dst_ref=x_buf.at[t],
                send_sem=x_ssem.at[t], recv_sem=x_rsem.at[t],
                device_id=(right,), device_id_type=pl.DeviceIdType.MESH)

        def a_send(t):
            return pltpu.make_async_remote_copy(
                src_ref=acc_buf.at[t - 1],
                dst_ref=acc_buf.at[t],
                send_sem=a_ssem.at[t - 1], recv_sem=a_rsem.at[t - 1],
                device_id=(right,), device_id_type=pl.DeviceIdType.MESH)

        pending = []
        x_rdmas = [x_send(t) for t in range(N_DEV - 1)]
        a_rdmas = [None] + [a_send(t) for t in range(1, N_DEV)]

        x_rdmas[0].start()
        pending.append(x_rdmas[0])
        own_scr[...] = compute_contrib(x_ref[...])

        for t in range(N_DEV - 1):
            x_rdmas[t].wait_recv()
            if t + 1 < N_DEV - 1:
                x_rdmas[t + 1].start()
                pending.append(x_rdmas[t + 1])

            c = compute_contrib(x_buf[t])
            if t == 0:
                acc_buf[0] = c.astype(jnp.bfloat16)
            else:
                a_rdmas[t].wait_recv()
                acc_buf[t] = (acc_buf[t].astype(jnp.float32)
                              + c).astype(jnp.bfloat16)
            a_rdmas[t + 1].start()
            pending.append(a_rdmas[t + 1])

        a_rdmas[N_DEV - 1].wait_recv()
        out_ref[...] = acc_buf[N_DEV - 1].astype(jnp.float32) + own_scr[...]

        for r in pending:
            r.wait_send()

    out = pl.pallas_call(
        body,
        out_shape=jax.ShapeDtypeStruct((sq, dm), jnp.float32),
        in_specs=[pl.BlockSpec(memory_space=pltpu.VMEM)] * 3
        + [pl.BlockSpec(memory_space=pltpu.MemorySpace.HBM)] * 2,
        out_specs=pl.BlockSpec(memory_space=pltpu.VMEM),
        scratch_shapes=[
            pltpu.VMEM((H_PER, skv, DH), jnp.bfloat16),
            pltpu.VMEM((H_PER, skv, DH), jnp.bfloat16),
            pltpu.VMEM((4, skv, DH), jnp.float32),
            pltpu.VMEM((N_DEV - 1, sq, dm), jnp.bfloat16),
            pltpu.VMEM((N_DEV, sq, dm), jnp.bfloat16),
            pltpu.VMEM((sq, dm), jnp.bfloat16),
            pltpu.VMEM((sq, dm), jnp.bfloat16),
            pltpu.VMEM((sq, dm), jnp.float32),
            pltpu.SemaphoreType.DMA((4,)),
            pltpu.SemaphoreType.DMA((N_DEV - 1,)),
            pltpu.SemaphoreType.DMA((N_DEV - 1,)),
            pltpu.SemaphoreType.DMA((N_DEV - 1,)),
            pltpu.SemaphoreType.DMA((N_DEV - 1,)),
        ],
        compiler_params=pltpu.CompilerParams(
            collective_id=0, vmem_limit_bytes=100 * 1024 * 1024),
    )(x2, wq, wo, K_ext, V_ext)

    return out.reshape(1, sq, dm)


# device time: 121720 ns/iter; 1.6477x vs baseline; 1.0539x over previous
import jax
import jax.numpy as jnp
from jax import lax
from jax.experimental import pallas as pl
from jax.experimental.pallas import tpu as pltpu

N_DEV = 8
H_PER = 8
DH = 128
SCALE = 0.08838834764831843


def kernel(x, Wq, Wo, K_ext, V_ext):
    _, sq, dm = x.shape
    skv = K_ext.shape[1]

    x2 = x[0].astype(jnp.bfloat16)
    wq = Wq.astype(jnp.bfloat16)
    wo = Wo.astype(jnp.bfloat16)

    def body(x_ref, wq_ref, wo_ref, k_any, v_any, out_ref,
             k_ref, v_ref, stage, x_buf, acc_buf, q_scr, o_scr, own_scr,
             kv_sem, x_ssem, x_rsem, a_ssem, a_rsem):
        my = lax.axis_index("i")
        left = lax.rem(my + N_DEV - 1, N_DEV)
        right = lax.rem(my + 1, N_DEV)

        barrier = pltpu.get_barrier_semaphore()
        for nbr in (left, right):
            pl.semaphore_signal(barrier, inc=1, device_id=(nbr,),
                                device_id_type=pl.DeviceIdType.MESH)
        pl.semaphore_wait(barrier, 2)

        N_KV = 2 * H_PER

        def kv_dma(idx):
            src = k_any if idx % 2 == 0 else v_any
            return pltpu.make_async_copy(
                src.at[0, :, my * H_PER + idx // 2, :],
                stage.at[idx % 4],
                kv_sem.at[idx % 4])

        for idx in range(4):
            kv_dma(idx).start()

        def land_kv(idx):
            kv_dma(idx).wait()
            tgt = k_ref if idx % 2 == 0 else v_ref
            tgt[idx // 2] = stage[idx % 4].astype(jnp.bfloat16)
            if idx + 4 < N_KV:
                kv_dma(idx + 4).start()

        def compute_contrib(x_src, landing=False):
            q = lax.dot_general(x_src, wq_ref[...], (((1,), (0,)), ((), ())),
                                preferred_element_type=jnp.float32) * SCALE
            q_scr[...] = q.astype(jnp.bfloat16)

            for h in range(H_PER):
                if landing:
                    land_kv(2 * h)
                    land_kv(2 * h + 1)
                qh = q_scr[:, h * DH:(h + 1) * DH]
                kh = k_ref[h]
                s = lax.dot_general(qh, kh, (((1,), (1,)), ((), ())),
                                    preferred_element_type=jnp.float32)
                p = jnp.exp(s)
                l = jnp.sum(p, axis=1, keepdims=True)
                oh = lax.dot_general(p.astype(jnp.bfloat16), v_ref[h],
                                     (((1,), (0,)), ((), ())),
                                     preferred_element_type=jnp.float32)
                o_scr[:, h * DH:(h + 1) * DH] = (oh / l).astype(jnp.bfloat16)
            return lax.dot_general(o_scr[...], wo_ref[...],
                                   (((1,), (0,)), ((), ())),
                                   preferred_element_type=jnp.float32)

        def x_send(t):
            return pltpu.make_async_remote_copy(
                src_ref=x_ref if t == 0 else x_buf.at[t - 1],
                dst_ref=x_buf.at[t],
                send_sem=x_ssem.at[t], recv_sem=x_rsem.at[t],
                device_id=(right,), device_id_type=pl.DeviceIdType.MESH)

        def a_send(t):
            return pltpu.make_async_remote_copy(
                src_ref=acc_buf.at[t - 1],
                dst_ref=acc_buf.at[t],
                send_sem=a_ssem.at[t - 1], recv_sem=a_rsem.at[t - 1],
                device_id=(right,), device_id_type=pl.DeviceIdType.MESH)

        pending = []
        x_rdmas = [x_send(t) for t in range(N_DEV - 1)]
        a_rdmas = [None] + [a_send(t) for t in range(1, N_DEV)]

        x_rdmas[0].start()
        pending.append(x_rdmas[0])
        own_scr[...] = compute_contrib(x_ref[...], landing=True)

        for t in range(N_DEV - 1):
            x_rdmas[t].wait_recv()
            if t + 1 < N_DEV - 1:
                x_rdmas[t + 1].start()
                pending.append(x_rdmas[t + 1])

            c = compute_contrib(x_buf[t])
            if t == 0:
                acc_buf[0] = c.astype(jnp.bfloat16)
            else:
                a_rdmas[t].wait_recv()
                acc_buf[t] = (acc_buf[t].astype(jnp.float32)
                              + c).astype(jnp.bfloat16)
            a_rdmas[t + 1].start()
            pending.append(a_rdmas[t + 1])

        a_rdmas[N_DEV - 1].wait_recv()
        out_ref[...] = acc_buf[N_DEV - 1].astype(jnp.float32) + own_scr[...]

        for r in pending:
            r.wait_send()

    out = pl.pallas_call(
        body,
        out_shape=jax.ShapeDtypeStruct((sq, dm), jnp.float32),
        in_specs=[pl.BlockSpec(memory_space=pltpu.VMEM)] * 3
        + [pl.BlockSpec(memory_space=pltpu.MemorySpace.HBM)] * 2,
        out_specs=pl.BlockSpec(memory_space=pltpu.VMEM),
        scratch_shapes=[
            pltpu.VMEM((H_PER, skv, DH), jnp.bfloat16),
            pltpu.VMEM((H_PER, skv, DH), jnp.bfloat16),
            pltpu.VMEM((4, skv, DH), jnp.float32),
            pltpu.VMEM((N_DEV - 1, sq, dm), jnp.bfloat16),
            pltpu.VMEM((N_DEV, sq, dm), jnp.bfloat16),
            pltpu.VMEM((sq, dm), jnp.bfloat16),
            pltpu.VMEM((sq, dm), jnp.bfloat16),
            pltpu.VMEM((sq, dm), jnp.float32),
            pltpu.SemaphoreType.DMA((4,)),
            pltpu.SemaphoreType.DMA((N_DEV - 1,)),
            pltpu.SemaphoreType.DMA((N_DEV - 1,)),
            pltpu.SemaphoreType.DMA((N_DEV - 1,)),
            pltpu.SemaphoreType.DMA((N_DEV - 1,)),
        ],
        compiler_params=pltpu.CompilerParams(
            collective_id=0, vmem_limit_bytes=100 * 1024 * 1024),
    )(x2, wq, wo, K_ext, V_ext)

    return out.reshape(1, sq, dm)
